# baseline (speedup 1.0000x reference)
import functools

import jax
import jax.numpy as jnp
from jax import lax
from jax.experimental import pallas as pl
from jax.experimental.pallas import tpu as pltpu
from jax.experimental.pallas import tpu_sc as plsc

N = 10000
NP = 10112
E = 320000
B = 256
NSUB = 16
ECHUNK = 80
NJ = 250
IBLK = 10
NBLK = NJ // IBLK
EPAD = NSUB * NJ * ECHUNK
ESUB = E // NSUB
DP = 80
D2P = 160
D3P = 320
OD = 128
XTP = 1024
ROWS_SUB = NP // NSUB
RCH = 8
NEG = -3.0e38

_i32 = jnp.int32
_f32 = jnp.float32


def _iota16():
    return lax.iota(_i32, 16)


def _make_deg_kernel():
    mesh = plsc.VectorSubcoreMesh(core_axis_name="c", subcore_axis_name="s")

    @functools.partial(
        pl.kernel,
        mesh=mesh,
        out_type=jax.ShapeDtypeStruct((2 * NP, 16), _f32),
        scratch_types=[
            pltpu.VMEM_SHARED((NP, 16), _f32),
            pltpu.VMEM((IBLK * ECHUNK,), _i32),
            pltpu.VMEM((ECHUNK, 16), _f32),
        ],
        compiler_params=pltpu.CompilerParams(use_tc_tiling_on_sc=False),
    )
    def deg(dst_hbm, zeros_hbm, ones_hbm, out_hbm, acc, dstb, ones):
        c = lax.axis_index("c")
        s = lax.axis_index("s")
        pltpu.sync_copy(ones_hbm, ones)
        pltpu.sync_copy(zeros_hbm.at[pl.ds(ROWS_SUB * s, ROWS_SUB)],
                        acc.at[pl.ds(ROWS_SUB * s, ROWS_SUB)])
        plsc.subcore_barrier()

        def blk(b, _):
            pltpu.sync_copy(
                dst_hbm.at[pl.ds(c * E + s * ESUB + b * IBLK * ECHUNK,
                                 IBLK * ECHUNK)], dstb)
            for j in range(IBLK):
                pltpu.sync_copy(ones,
                                acc.at[dstb.at[pl.ds(j * ECHUNK, ECHUNK)]],
                                add=True)
            return 0

        lax.fori_loop(0, NBLK, blk, 0)
        plsc.subcore_barrier()
        pltpu.sync_copy(acc.at[pl.ds(ROWS_SUB * s, ROWS_SUB)],
                        out_hbm.at[pl.ds(c * NP + ROWS_SUB * s, ROWS_SUB)])

    return deg


def _make_edge_kernel(F, ech, iblk):
    mesh = plsc.VectorSubcoreMesh(core_axis_name="c", subcore_axis_name="s")
    nj = EPAD // NSUB // ech
    nblk = nj // iblk

    @functools.partial(
        pl.kernel,
        mesh=mesh,
        out_type=jax.ShapeDtypeStruct((2 * NP, F), _f32),
        scratch_types=[
            pltpu.VMEM_SHARED((NP, F), _f32),
            pltpu.VMEM((iblk * ech,), _i32),
            pltpu.VMEM((iblk * ech,), _i32),
            pltpu.VMEM((ech, F), _f32),
            pltpu.VMEM((ech, F), _f32),
            pltpu.SemaphoreType.DMA,
            pltpu.SemaphoreType.DMA,
            pltpu.SemaphoreType.DMA,
            pltpu.SemaphoreType.DMA,
        ],
        compiler_params=pltpu.CompilerParams(use_tc_tiling_on_sc=False),
    )
    def edge(y_hbm, src_hbm, dst_hbm, zeros_hbm, out_hbm, acc, srcb, dstb,
             rows0, rows1, g0, g1, s0, s1):
        c = lax.axis_index("c")
        s = lax.axis_index("s")
        base = ROWS_SUB * s
        pltpu.sync_copy(zeros_hbm.at[pl.ds(base, ROWS_SUB)],
                        acc.at[pl.ds(base, ROWS_SUB)])
        plsc.subcore_barrier()
        bufs = (rows0, rows1)
        gsems = (g0, g1)
        ssems = (s0, s1)

        def blk(b, _):
            off = c * E + s * ESUB + b * iblk * ech
            pltpu.sync_copy(src_hbm.at[pl.ds(off, iblk * ech)], srcb)
            pltpu.sync_copy(dst_hbm.at[pl.ds(off, iblk * ech)], dstb)
            hg = {0: pltpu.async_copy(y_hbm.at[srcb.at[pl.ds(0, ech)]],
                                      bufs[0], gsems[0])}
            hs = {}
            for j in range(iblk):
                nb = (j + 1) % 2
                if j + 1 < iblk:
                    if j >= 1:
                        hs[j - 1].wait()
                    hg[j + 1] = pltpu.async_copy(
                        y_hbm.at[srcb.at[pl.ds((j + 1) * ech, ech)]],
                        bufs[nb], gsems[nb])
                hg[j].wait()
                hs[j] = pltpu.async_copy(
                    bufs[j % 2], acc.at[dstb.at[pl.ds(j * ech, ech)]],
                    ssems[j % 2], add=True)
            hs[iblk - 2].wait()
            hs[iblk - 1].wait()
            return 0

        lax.fori_loop(0, nblk, blk, 0)
        plsc.subcore_barrier()
        pltpu.sync_copy(acc.at[pl.ds(base, ROWS_SUB)],
                        out_hbm.at[pl.ds(c * NP + base, ROWS_SUB)])

    return edge


def _make_segmax_kernel():
    mesh = plsc.VectorSubcoreMesh(core_axis_name="c", subcore_axis_name="s")
    F = D2P
    SEGS_SUB = B // NSUB
    RC = 79
    NCH = ROWS_SUB // RC

    @functools.partial(
        pl.kernel,
        mesh=mesh,
        out_type=jax.ShapeDtypeStruct((2, 2 * B * F), _f32),
        scratch_types=[
            pltpu.VMEM_SHARED((NSUB, B * F), _f32),
            pltpu.VMEM((B * F,), _f32),
            pltpu.VMEM((RC * F,), _f32),
            pltpu.VMEM((RC * F,), _f32),
            pltpu.VMEM((ROWS_SUB,), _i32),
            pltpu.VMEM((SEGS_SUB * F,), _f32),
            pltpu.VMEM((SEGS_SUB * F,), _f32),
            pltpu.SemaphoreType.DMA,
            pltpu.SemaphoreType.DMA,
        ],
        compiler_params=pltpu.CompilerParams(use_tc_tiling_on_sc=False,
                                             needs_layout_passes=False),
    )
    def segmax(ha_hbm, hb_hbm, batch_hbm, neg_hbm, out_hbm, stage, table,
               rbuf0, rbuf1, bbuf, cbuf, macc, semA, semB):
        c = lax.axis_index("c")
        s = lax.axis_index("s")
        cols = F // 16
        pltpu.sync_copy(batch_hbm.at[c, s], bbuf)
        base = (c * NP + ROWS_SUB * s) * F
        seg0 = SEGS_SUB * F * s
        bufs = (rbuf0, rbuf1)
        sems = (semA, semB)
        negv = jnp.full((16,), NEG, _f32)

        for hf, h_hbm in enumerate((ha_hbm, hb_hbm)):
            pltpu.sync_copy(neg_hbm, table)
            seg_init = plsc.load_gather(bbuf, [jnp.full((16,), 0, _i32)])
            carry = (seg_init,) + (negv,) * cols
            pend = {0: pltpu.async_copy(h_hbm.at[pl.ds(base, RC * F)],
                                        bufs[0], sems[0])}
            for t in range(NCH):
                if t + 1 < NCH:
                    pend[t + 1] = pltpu.async_copy(
                        h_hbm.at[pl.ds(base + RC * F * (t + 1), RC * F)],
                        bufs[(t + 1) % 2], sems[(t + 1) % 2])
                pend[t].wait()
                rb_ref = bufs[t % 2]

                def rowstep(r, carry, rb_ref=rb_ref, t=t):
                    prev_seg = carry[0]
                    accs = carry[1:]
                    seg = plsc.load_gather(
                        bbuf, [jnp.full((16,), RC * t + r, _i32)])
                    rb = r * F
                    rv = tuple(rb_ref[pl.ds(rb + 16 * f, 16)]
                               for f in range(cols))
                    same = jnp.max(jnp.where(seg == prev_seg, 0, 1)) == 0

                    def same_fn():
                        return (prev_seg,) + tuple(
                            jnp.maximum(accs[f], rv[f])
                            for f in range(cols))

                    def flush_fn():
                        pb = prev_seg * F
                        for f in range(cols):
                            plsc.store_scatter(table,
                                               [pb + 16 * f + _iota16()],
                                               accs[f])
                        return (seg,) + rv

                    return lax.cond(same, same_fn, flush_fn)

                carry = lax.fori_loop(0, RC, rowstep, carry)
            pb = carry[0] * F
            for f in range(cols):
                plsc.store_scatter(table, [pb + 16 * f + _iota16()],
                                   carry[1 + f])
            pltpu.sync_copy(table, stage.at[s])
            plsc.subcore_barrier()
            pltpu.sync_copy(stage.at[0, pl.ds(seg0, SEGS_SUB * F)], macc)
            for t in range(1, NSUB):
                pltpu.sync_copy(stage.at[t, pl.ds(seg0, SEGS_SUB * F)], cbuf)

                def mstep(i, _):
                    off = 16 * i
                    macc[pl.ds(off, 16)] = jnp.maximum(
                        macc[pl.ds(off, 16)], cbuf[pl.ds(off, 16)])
                    return 0

                lax.fori_loop(0, SEGS_SUB * F // 16, mstep, 0)
            pltpu.sync_copy(macc,
                            out_hbm.at[hf, pl.ds(c * B * F + seg0,
                                                 SEGS_SUB * F)])
            plsc.subcore_barrier()

    return segmax


_RB = 2528
_HI = jax.lax.Precision.HIGHEST


def _dinv_block(cnt_blk, pid):
    rows = lax.broadcasted_iota(_i32, (_RB, 1), 0) + pid * _RB
    local = rows % NP
    d = lax.rsqrt(cnt_blk[:, 0:1] + 1.0)
    return jnp.where(local < N, d, 0.0)


def _tc_scale_body(x_ref, cnt_ref, o_ref):
    o_ref[...] = x_ref[...] * _dinv_block(cnt_ref[...], pl.program_id(0))


def _tc_scale(x, cnt):
    return pl.pallas_call(
        _tc_scale_body,
        grid=(8,),
        in_specs=[pl.BlockSpec((_RB, DP), lambda i: (i, 0)),
                  pl.BlockSpec((_RB, 16), lambda i: (i, 0))],
        out_specs=pl.BlockSpec((_RB, DP), lambda i: (i, 0)),
        out_shape=jax.ShapeDtypeStruct((2 * NP, DP), _f32),
    )(x, cnt)


def _tc_layer_body(scale_out, acc_ref, y_ref, cnt_ref, w_ref, b_ref, *o_ref):
    if scale_out:
        o_ref = o_ref[0]
    dinv = _dinv_block(cnt_ref[...], pl.program_id(0))
    z = (acc_ref[...] + y_ref[...]) * dinv
    h = lax.dot_general(z, w_ref[...], (((1,), (0,)), ((), ())),
                        precision=_HI, preferred_element_type=_f32)
    h = jnp.maximum(h + b_ref[...], 0.0)
    if scale_out:
        o_ref[...] = h * dinv
    else:
        h = jnp.where(dinv > 0.0, h, NEG)
        o_ref[0][...] = h[:, :D2P]
        o_ref[1][...] = h[:, D2P:]


def _tc_layer(acc, y, cnt, w, b, scale_out):
    fin, fout = w.shape
    return pl.pallas_call(
        functools.partial(_tc_layer_body, scale_out),
        grid=(8,),
        in_specs=[pl.BlockSpec((_RB, fin), lambda i: (i, 0)),
                  pl.BlockSpec((_RB, fin), lambda i: (i, 0)),
                  pl.BlockSpec((_RB, 16), lambda i: (i, 0)),
                  pl.BlockSpec((fin, fout), lambda i: (0, 0)),
                  pl.BlockSpec((1, fout), lambda i: (0, 0))],
        out_specs=(pl.BlockSpec((_RB, fout), lambda i: (i, 0))
                   if scale_out else
                   [pl.BlockSpec((_RB, D2P), lambda i: (i, 0))] * 2),
        out_shape=(jax.ShapeDtypeStruct((2 * NP, fout), _f32)
                   if scale_out else
                   [jax.ShapeDtypeStruct((2 * NP, D2P), _f32)] * 2),
    )(acc, y, cnt, w, b)


def _mm(a, b):
    return lax.dot_general(a, b, (((1,), (0,)), ((), ())),
                           precision=_HI, preferred_element_type=_f32)


def _tail_body(g_ref, cell_ref, wg1, bg1, wg2, bg2, wr1, br1, wr2, br2, wr3,
               br3, wf1, bf1, wf2, bf2, wo, bo, ap, o_ref):
    g = jnp.maximum(_mm(g_ref[...], wg1[...]) + bg1[...], 0.0)
    g = _mm(g, wg2[...]) + bg2[...]
    g1 = g[0:B]
    g2 = g[B:2 * B]
    cc = jnp.maximum(_mm(cell_ref[...], wr1[...]) + br1[...], 0.0)
    cc = jnp.maximum(_mm(cc, wr2[...]) + br2[...], 0.0)
    cc = _mm(cc, wr3[...]) + br3[...]
    ss = (jnp.sum(g1 * g1, axis=1, keepdims=True)
          + jnp.sum(g2 * g2, axis=1, keepdims=True)
          + jnp.sum(cc * cc, axis=1, keepdims=True))
    inv = 1.0 / jnp.maximum(jnp.sqrt(ss), 1e-12)
    w = wf1[...]
    t = (_mm(g1, w[0:OD]) + _mm(g2, w[OD:2 * OD])
         + _mm(cc, w[2 * OD:3 * OD])) * inv + bf1[...]
    a = ap[0, 0]
    t = jnp.where(t >= 0, t, a * t)
    u = _mm(t, wf2[...]) + bf2[...]
    u = jnp.where(u >= 0, u, a * u)
    o = _mm(u, wo[...]) + bo[...]
    o_ref[...] = 1.0 / (1.0 + jnp.exp(-o))


def _pad2(a, r, c):
    return jnp.pad(a, ((0, r - a.shape[0]), (0, c - a.shape[1])))


_deg_call = _make_deg_kernel()
_edge80 = _make_edge_kernel(DP, 80, 10)
_edge160 = _make_edge_kernel(D2P, 80, 10)
_segmax_call = _make_segmax_kernel()


def kernel(x1, edge_index1, x2, edge_index2, cell, batch1, batch2, W1, b1,
           W2, b2, W3, b3, Wg1, bg1, Wg2, bg2, Wr1, br1, Wr2, br2, Wr3, br3,
           Wf1, bf1, Wf2, bf2, Wo, bo, a_prelu):
    x = jnp.concatenate([_pad2(x1, NP, DP), _pad2(x2, NP, DP)])
    src = jnp.concatenate([edge_index1[0], edge_index2[0] + NP])
    dst = jnp.concatenate([edge_index1[1], edge_index2[1]])
    batch = jnp.stack([
        jnp.pad(batch1, (0, NP - N)).reshape(NSUB, ROWS_SUB),
        jnp.pad(batch2, (0, NP - N)).reshape(NSUB, ROWS_SUB),
    ])
    w1 = _pad2(W1, DP, DP)
    w2 = _pad2(W2, DP, D2P)
    w3 = _pad2(W3, D2P, D3P)
    b1p = jnp.pad(b1, (0, DP - 78)).reshape(1, DP)
    b2p = jnp.pad(b2, (0, D2P - 156)).reshape(1, D2P)
    b3p = jnp.pad(b3, (0, D3P - 312)).reshape(1, D3P)
    wg1 = _pad2(Wg1, D3P, D2P)
    bg1p = jnp.pad(bg1, (0, D2P - 156)).reshape(1, D2P)
    wg2 = _pad2(Wg2, D2P, OD)
    cellT = _pad2(jnp.transpose(cell), B, XTP)
    wr1 = _pad2(Wr1, XTP, 512)

    zeros16 = jnp.zeros((NP, 16), _f32)
    ones128 = jnp.ones((ECHUNK, 16), _f32)
    zeros80 = jnp.zeros((NP, DP), _f32)
    zeros160 = jnp.zeros((NP, D2P), _f32)
    negarr = jnp.full((B * D2P,), NEG, _f32)

    cnt = _deg_call(dst, zeros16, ones128)
    y = _tc_scale(x, cnt)
    acc = _edge80(y, src, dst, zeros80)
    y = _tc_layer(acc, y, cnt, w1, b1p, True)
    acc = _edge80(y, src, dst, zeros80)
    y = _tc_layer(acc, y, cnt, w2, b2p, True)
    acc = _edge160(y, src, dst, zeros160)
    h3a, h3b = _tc_layer(acc, y, cnt, w3, b3p, False)
    gh = _segmax_call(h3a.reshape(-1), h3b.reshape(-1), batch, negarr)
    g = jnp.concatenate([gh[0].reshape(2 * B, D2P),
                         gh[1].reshape(2 * B, D2P)], axis=1)

    out = pl.pallas_call(
        _tail_body,
        out_shape=jax.ShapeDtypeStruct((B, 1), _f32),
    )(g, cellT, wg1, bg1p, wg2, bg2.reshape(1, OD), wr1,
      br1.reshape(1, 512), Wr2, br2.reshape(1, 256), Wr3,
      br3.reshape(1, OD), Wf1, bf1.reshape(1, 512), Wf2,
      bf2.reshape(1, OD), Wo, bo.reshape(1, 1),
      a_prelu.reshape(1, 1))
    return out


# speedup vs baseline: 1.0634x; 1.0634x over previous
import functools

import jax
import jax.numpy as jnp
from jax import lax
from jax.experimental import pallas as pl
from jax.experimental.pallas import tpu as pltpu
from jax.experimental.pallas import tpu_sc as plsc

N = 10000
NP = 10112
E = 320000
B = 256
NSUB = 16
ECHUNK = 128
NJ = 160
IBLK = 8
NBLK = NJ // IBLK
EPAD = NSUB * NJ * ECHUNK
ESUB = EPAD // NSUB
DP = 80
D2P = 160
D3P = 320
OD = 128
XTP = 1024
ROWS_SUB = NP // NSUB
RCH = 8
NEG = -3.0e38

_i32 = jnp.int32
_f32 = jnp.float32


def _iota16():
    return lax.iota(_i32, 16)


def _make_deg_kernel():
    mesh = plsc.VectorSubcoreMesh(core_axis_name="c", subcore_axis_name="s")

    @functools.partial(
        pl.kernel,
        mesh=mesh,
        out_type=jax.ShapeDtypeStruct((2 * NP, 16), _f32),
        scratch_types=[
            pltpu.VMEM_SHARED((NP, 16), _f32),
            pltpu.VMEM((IBLK * ECHUNK,), _i32),
            pltpu.VMEM((ECHUNK, 16), _f32),
        ],
        compiler_params=pltpu.CompilerParams(use_tc_tiling_on_sc=False),
    )
    def deg(dst_hbm, zeros_hbm, ones_hbm, out_hbm, acc, dstb, ones):
        c = lax.axis_index("c")
        s = lax.axis_index("s")
        pltpu.sync_copy(ones_hbm, ones)
        pltpu.sync_copy(zeros_hbm.at[pl.ds(ROWS_SUB * s, ROWS_SUB)],
                        acc.at[pl.ds(ROWS_SUB * s, ROWS_SUB)])
        plsc.subcore_barrier()

        def blk(b, _):
            pltpu.sync_copy(
                dst_hbm.at[pl.ds(c * EPAD + s * ESUB + b * IBLK * ECHUNK,
                                 IBLK * ECHUNK)], dstb)
            for j in range(IBLK):
                pltpu.sync_copy(ones,
                                acc.at[dstb.at[pl.ds(j * ECHUNK, ECHUNK)]],
                                add=True)
            return 0

        lax.fori_loop(0, NBLK, blk, 0)
        plsc.subcore_barrier()
        pltpu.sync_copy(acc.at[pl.ds(ROWS_SUB * s, ROWS_SUB)],
                        out_hbm.at[pl.ds(c * NP + ROWS_SUB * s, ROWS_SUB)])

    return deg


def _make_edge_kernel(F, ech, iblk):
    mesh = plsc.VectorSubcoreMesh(core_axis_name="c", subcore_axis_name="s")
    nj = EPAD // NSUB // ech
    nblk = nj // iblk

    @functools.partial(
        pl.kernel,
        mesh=mesh,
        out_type=jax.ShapeDtypeStruct((2 * NP, F), _f32),
        scratch_types=[
            pltpu.VMEM_SHARED((NP, F), _f32),
            pltpu.VMEM((iblk * ech,), _i32),
            pltpu.VMEM((iblk * ech,), _i32),
            pltpu.VMEM((ech, F), _f32),
            pltpu.VMEM((ech, F), _f32),
            pltpu.SemaphoreType.DMA,
            pltpu.SemaphoreType.DMA,
            pltpu.SemaphoreType.DMA,
            pltpu.SemaphoreType.DMA,
        ],
        compiler_params=pltpu.CompilerParams(use_tc_tiling_on_sc=False),
    )
    def edge(y_hbm, src_hbm, dst_hbm, zeros_hbm, out_hbm, acc, srcb, dstb,
             rows0, rows1, g0, g1, s0, s1):
        c = lax.axis_index("c")
        s = lax.axis_index("s")
        base = ROWS_SUB * s
        pltpu.sync_copy(zeros_hbm.at[pl.ds(base, ROWS_SUB)],
                        acc.at[pl.ds(base, ROWS_SUB)])
        plsc.subcore_barrier()
        bufs = (rows0, rows1)
        gsems = (g0, g1)
        ssems = (s0, s1)

        def blk(b, _):
            off = c * EPAD + s * ESUB + b * iblk * ech
            pltpu.sync_copy(src_hbm.at[pl.ds(off, iblk * ech)], srcb)
            pltpu.sync_copy(dst_hbm.at[pl.ds(off, iblk * ech)], dstb)
            hg = {0: pltpu.async_copy(y_hbm.at[srcb.at[pl.ds(0, ech)]],
                                      bufs[0], gsems[0])}
            hs = {}
            for j in range(iblk):
                nb = (j + 1) % 2
                if j + 1 < iblk:
                    if j >= 1:
                        hs[j - 1].wait()
                    hg[j + 1] = pltpu.async_copy(
                        y_hbm.at[srcb.at[pl.ds((j + 1) * ech, ech)]],
                        bufs[nb], gsems[nb])
                hg[j].wait()
                hs[j] = pltpu.async_copy(
                    bufs[j % 2], acc.at[dstb.at[pl.ds(j * ech, ech)]],
                    ssems[j % 2], add=True)
            hs[iblk - 2].wait()
            hs[iblk - 1].wait()
            return 0

        lax.fori_loop(0, nblk, blk, 0)
        plsc.subcore_barrier()
        pltpu.sync_copy(acc.at[pl.ds(base, ROWS_SUB)],
                        out_hbm.at[pl.ds(c * NP + base, ROWS_SUB)])

    return edge


def _make_segmax_kernel():
    mesh = plsc.VectorSubcoreMesh(core_axis_name="c", subcore_axis_name="s")
    F = D2P
    SEGS_SUB = B // NSUB
    RC = 79
    NCH = ROWS_SUB // RC

    @functools.partial(
        pl.kernel,
        mesh=mesh,
        out_type=jax.ShapeDtypeStruct((2, 2 * B * F), _f32),
        scratch_types=[
            pltpu.VMEM_SHARED((NSUB, B * F), _f32),
            pltpu.VMEM((B * F,), _f32),
            pltpu.VMEM((RC * F,), _f32),
            pltpu.VMEM((RC * F,), _f32),
            pltpu.VMEM((ROWS_SUB,), _i32),
            pltpu.VMEM((SEGS_SUB * F,), _f32),
            pltpu.VMEM((SEGS_SUB * F,), _f32),
            pltpu.SemaphoreType.DMA,
            pltpu.SemaphoreType.DMA,
        ],
        compiler_params=pltpu.CompilerParams(use_tc_tiling_on_sc=False,
                                             needs_layout_passes=False),
    )
    def segmax(ha_hbm, hb_hbm, batch_hbm, neg_hbm, out_hbm, stage, table,
               rbuf0, rbuf1, bbuf, cbuf, macc, semA, semB):
        c = lax.axis_index("c")
        s = lax.axis_index("s")
        cols = F // 16
        pltpu.sync_copy(batch_hbm.at[c, s], bbuf)
        base = (c * NP + ROWS_SUB * s) * F
        seg0 = SEGS_SUB * F * s
        bufs = (rbuf0, rbuf1)
        sems = (semA, semB)
        negv = jnp.full((16,), NEG, _f32)

        for hf, h_hbm in enumerate((ha_hbm, hb_hbm)):
            pltpu.sync_copy(neg_hbm, table)
            seg_init = plsc.load_gather(bbuf, [jnp.full((16,), 0, _i32)])
            carry = (seg_init,) + (negv,) * cols
            pend = {0: pltpu.async_copy(h_hbm.at[pl.ds(base, RC * F)],
                                        bufs[0], sems[0])}
            for t in range(NCH):
                if t + 1 < NCH:
                    pend[t + 1] = pltpu.async_copy(
                        h_hbm.at[pl.ds(base + RC * F * (t + 1), RC * F)],
                        bufs[(t + 1) % 2], sems[(t + 1) % 2])
                pend[t].wait()
                rb_ref = bufs[t % 2]

                def rowstep(r, carry, rb_ref=rb_ref, t=t):
                    prev_seg = carry[0]
                    accs = carry[1:]
                    seg = plsc.load_gather(
                        bbuf, [jnp.full((16,), RC * t + r, _i32)])
                    rb = r * F
                    rv = tuple(rb_ref[pl.ds(rb + 16 * f, 16)]
                               for f in range(cols))
                    same = jnp.max(jnp.where(seg == prev_seg, 0, 1)) == 0

                    def same_fn():
                        return (prev_seg,) + tuple(
                            jnp.maximum(accs[f], rv[f])
                            for f in range(cols))

                    def flush_fn():
                        pb = prev_seg * F
                        for f in range(cols):
                            plsc.store_scatter(table,
                                               [pb + 16 * f + _iota16()],
                                               accs[f])
                        return (seg,) + rv

                    return lax.cond(same, same_fn, flush_fn)

                carry = lax.fori_loop(0, RC, rowstep, carry)
            pb = carry[0] * F
            for f in range(cols):
                plsc.store_scatter(table, [pb + 16 * f + _iota16()],
                                   carry[1 + f])
            pltpu.sync_copy(table, stage.at[s])
            plsc.subcore_barrier()
            pltpu.sync_copy(stage.at[0, pl.ds(seg0, SEGS_SUB * F)], macc)
            for t in range(1, NSUB):
                pltpu.sync_copy(stage.at[t, pl.ds(seg0, SEGS_SUB * F)], cbuf)

                def mstep(i, _):
                    for u in range(4):
                        off = 64 * i + 16 * u
                        macc[pl.ds(off, 16)] = jnp.maximum(
                            macc[pl.ds(off, 16)], cbuf[pl.ds(off, 16)])
                    return 0

                lax.fori_loop(0, SEGS_SUB * F // 64, mstep, 0)
            pltpu.sync_copy(macc,
                            out_hbm.at[hf, pl.ds(c * B * F + seg0,
                                                 SEGS_SUB * F)])
            plsc.subcore_barrier()

    return segmax


_RB = 2528
_HI = jax.lax.Precision.HIGHEST


def _dinv_block(cnt_blk, pid):
    rows = lax.broadcasted_iota(_i32, (_RB, 1), 0) + pid * _RB
    local = rows % NP
    d = lax.rsqrt(cnt_blk[:, 0:1] + 1.0)
    return jnp.where(local < N, d, 0.0)


def _tc_scale_body(x_ref, cnt_ref, o_ref):
    o_ref[...] = x_ref[...] * _dinv_block(cnt_ref[...], pl.program_id(0))


def _tc_scale(x, cnt):
    return pl.pallas_call(
        _tc_scale_body,
        grid=(8,),
        in_specs=[pl.BlockSpec((_RB, DP), lambda i: (i, 0)),
                  pl.BlockSpec((_RB, 16), lambda i: (i, 0))],
        out_specs=pl.BlockSpec((_RB, DP), lambda i: (i, 0)),
        out_shape=jax.ShapeDtypeStruct((2 * NP, DP), _f32),
    )(x, cnt)


def _tc_layer_body(scale_out, acc_ref, y_ref, cnt_ref, w_ref, b_ref, *o_ref):
    if scale_out:
        o_ref = o_ref[0]
    dinv = _dinv_block(cnt_ref[...], pl.program_id(0))
    z = (acc_ref[...] + y_ref[...]) * dinv
    h = lax.dot_general(z, w_ref[...], (((1,), (0,)), ((), ())),
                        precision=_HI, preferred_element_type=_f32)
    h = jnp.maximum(h + b_ref[...], 0.0)
    if scale_out:
        o_ref[...] = h * dinv
    else:
        h = jnp.where(dinv > 0.0, h, NEG)
        o_ref[0][...] = h[:, :D2P]
        o_ref[1][...] = h[:, D2P:]


def _tc_layer(acc, y, cnt, w, b, scale_out):
    fin, fout = w.shape
    return pl.pallas_call(
        functools.partial(_tc_layer_body, scale_out),
        grid=(8,),
        in_specs=[pl.BlockSpec((_RB, fin), lambda i: (i, 0)),
                  pl.BlockSpec((_RB, fin), lambda i: (i, 0)),
                  pl.BlockSpec((_RB, 16), lambda i: (i, 0)),
                  pl.BlockSpec((fin, fout), lambda i: (0, 0)),
                  pl.BlockSpec((1, fout), lambda i: (0, 0))],
        out_specs=(pl.BlockSpec((_RB, fout), lambda i: (i, 0))
                   if scale_out else
                   [pl.BlockSpec((_RB, D2P), lambda i: (i, 0))] * 2),
        out_shape=(jax.ShapeDtypeStruct((2 * NP, fout), _f32)
                   if scale_out else
                   [jax.ShapeDtypeStruct((2 * NP, D2P), _f32)] * 2),
    )(acc, y, cnt, w, b)


def _mm(a, b):
    return lax.dot_general(a, b, (((1,), (0,)), ((), ())),
                           precision=_HI, preferred_element_type=_f32)


def _tail_body(g_ref, cell_ref, wg1, bg1, wg2, bg2, wr1, br1, wr2, br2, wr3,
               br3, wf1, bf1, wf2, bf2, wo, bo, ap, o_ref):
    g = jnp.maximum(_mm(g_ref[...], wg1[...]) + bg1[...], 0.0)
    g = _mm(g, wg2[...]) + bg2[...]
    g1 = g[0:B]
    g2 = g[B:2 * B]
    cc = jnp.maximum(_mm(cell_ref[...], wr1[...]) + br1[...], 0.0)
    cc = jnp.maximum(_mm(cc, wr2[...]) + br2[...], 0.0)
    cc = _mm(cc, wr3[...]) + br3[...]
    ss = (jnp.sum(g1 * g1, axis=1, keepdims=True)
          + jnp.sum(g2 * g2, axis=1, keepdims=True)
          + jnp.sum(cc * cc, axis=1, keepdims=True))
    inv = 1.0 / jnp.maximum(jnp.sqrt(ss), 1e-12)
    w = wf1[...]
    t = (_mm(g1, w[0:OD]) + _mm(g2, w[OD:2 * OD])
         + _mm(cc, w[2 * OD:3 * OD])) * inv + bf1[...]
    a = ap[0, 0]
    t = jnp.where(t >= 0, t, a * t)
    u = _mm(t, wf2[...]) + bf2[...]
    u = jnp.where(u >= 0, u, a * u)
    o = _mm(u, wo[...]) + bo[...]
    o_ref[...] = 1.0 / (1.0 + jnp.exp(-o))


def _pad2(a, r, c):
    return jnp.pad(a, ((0, r - a.shape[0]), (0, c - a.shape[1])))


_deg_call = _make_deg_kernel()
_edge80 = _make_edge_kernel(DP, 128, 8)
_edge160 = _make_edge_kernel(D2P, 80, 8)
_segmax_call = _make_segmax_kernel()


def kernel(x1, edge_index1, x2, edge_index2, cell, batch1, batch2, W1, b1,
           W2, b2, W3, b3, Wg1, bg1, Wg2, bg2, Wr1, br1, Wr2, br2, Wr3, br3,
           Wf1, bf1, Wf2, bf2, Wo, bo, a_prelu):
    x = jnp.concatenate([_pad2(x1, NP, DP), _pad2(x2, NP, DP)])
    padr = N + (jnp.arange(EPAD - E, dtype=_i32) % (NP - N))
    src = jnp.concatenate([edge_index1[0], padr,
                           edge_index2[0] + NP, padr + NP])
    dst = jnp.concatenate([edge_index1[1], padr, edge_index2[1], padr])
    batch = jnp.stack([
        jnp.pad(batch1, (0, NP - N)).reshape(NSUB, ROWS_SUB),
        jnp.pad(batch2, (0, NP - N)).reshape(NSUB, ROWS_SUB),
    ])
    w1 = _pad2(W1, DP, DP)
    w2 = _pad2(W2, DP, D2P)
    w3 = _pad2(W3, D2P, D3P)
    b1p = jnp.pad(b1, (0, DP - 78)).reshape(1, DP)
    b2p = jnp.pad(b2, (0, D2P - 156)).reshape(1, D2P)
    b3p = jnp.pad(b3, (0, D3P - 312)).reshape(1, D3P)
    wg1 = _pad2(Wg1, D3P, D2P)
    bg1p = jnp.pad(bg1, (0, D2P - 156)).reshape(1, D2P)
    wg2 = _pad2(Wg2, D2P, OD)
    cellT = _pad2(jnp.transpose(cell), B, XTP)
    wr1 = _pad2(Wr1, XTP, 512)

    zeros16 = jnp.zeros((NP, 16), _f32)
    ones128 = jnp.ones((ECHUNK, 16), _f32)
    zeros80 = jnp.zeros((NP, DP), _f32)
    zeros160 = jnp.zeros((NP, D2P), _f32)
    negarr = jnp.full((B * D2P,), NEG, _f32)

    cnt = _deg_call(dst, zeros16, ones128)
    y = _tc_scale(x, cnt)
    acc = _edge80(y, src, dst, zeros80)
    y = _tc_layer(acc, y, cnt, w1, b1p, True)
    acc = _edge80(y, src, dst, zeros80)
    y = _tc_layer(acc, y, cnt, w2, b2p, True)
    acc = _edge160(y, src, dst, zeros160)
    h3a, h3b = _tc_layer(acc, y, cnt, w3, b3p, False)
    gh = _segmax_call(h3a.reshape(-1), h3b.reshape(-1), batch, negarr)
    g = jnp.concatenate([gh[0].reshape(2 * B, D2P),
                         gh[1].reshape(2 * B, D2P)], axis=1)

    out = pl.pallas_call(
        _tail_body,
        out_shape=jax.ShapeDtypeStruct((B, 1), _f32),
    )(g, cellT, wg1, bg1p, wg2, bg2.reshape(1, OD), wr1,
      br1.reshape(1, 512), Wr2, br2.reshape(1, 256), Wr3,
      br3.reshape(1, OD), Wf1, bf1.reshape(1, 512), Wf2,
      bf2.reshape(1, OD), Wo, bo.reshape(1, 1),
      a_prelu.reshape(1, 1))
    return out
